# baseline (speedup 1.0000x reference)
import functools

import jax
import jax.numpy as jnp
from jax import lax
from jax.experimental import pallas as pl
from jax.experimental.pallas import tpu as pltpu
from jax.experimental.pallas import tpu_sc as plsc

_NUM_ORIG = 1000000
_NUM_NEW = 1000
_D = 64
_L = 16

_NC = 2
_NS = 16
_NW = _NC * _NS

_B_TOTAL = 16384 * 50
_B_PER_W = _B_TOTAL // _NW
_C = 512
_NCHUNK = _B_PER_W // _C
_IDXW = 128
_NDMA = _C // _IDXW
_NGRP = _C // _L


def _body(ids_hbm, w_orig_hbm, w_new_hbm, out_hbm,
          w_new_v, ids_v, idx_v, rows_v, flag_v, sem_g0, sem_g1, sem_w0,
          sem_w1):
    c_id = lax.axis_index("c")
    s_id = lax.axis_index("s")
    wid = s_id * _NC + c_id
    base = wid * _B_PER_W
    sem_g = (sem_g0, sem_g1)
    sem_w = (sem_w0, sem_w1)

    pltpu.sync_copy(w_new_hbm, w_new_v)

    iota = lax.iota(jnp.int32, _L)

    def gather_descs(b):
        return [
            pltpu.make_async_copy(
                w_orig_hbm.at[idx_v.at[j]],
                rows_v.at[b].at[pl.ds(j * _IDXW, _IDXW)],
                sem_g[b],
            )
            for j in range(_NDMA)
        ]

    def write_desc(b, pos):
        return pltpu.make_async_copy(
            rows_v.at[b], out_hbm.at[pl.ds(pos, _C)], sem_w[b])

    def prep(chunk, b):
        pos = base + chunk * _C
        pltpu.sync_copy(ids_hbm.at[pl.ds(pos, _C)], ids_v)
        flag_v[pl.ds(0, _L)] = jnp.zeros((_L,), jnp.int32)

        def safe_body(g, carry):
            idvec = ids_v[pl.ds(g * _L, _L)]
            mask = idvec >= _NUM_ORIG
            safe = jnp.where(mask, 0, idvec)
            gpr = _IDXW // _L
            idx_v[g // gpr, pl.ds((g % gpr) * _L, _L)] = safe
            plsc.store_scatter(flag_v, [jnp.full((_L,), 0, jnp.int32)],
                               jnp.full((_L,), 1, jnp.int32), mask=mask)
            return carry

        lax.fori_loop(0, _NGRP, safe_body, 0)
        for d in gather_descs(b):
            d.start()

    def fixup(b):
        def fix_body(g, carry):
            idvec = ids_v[pl.ds(g * _L, _L)]
            mask = idvec >= _NUM_ORIG
            has_new = plsc.all_reduce_population_count(mask)[0] > 0

            @pl.when(has_new)
            def _fix():
                new_ids = jnp.where(mask, idvec - _NUM_ORIG, 0)
                row_pos = g * _L + iota

                def col_body(col, carry2):
                    colv = jnp.full((_L,), col, jnp.int32)
                    x = plsc.load_gather(w_new_v, [new_ids, colv], mask=mask)
                    plsc.store_scatter(rows_v.at[b], [row_pos, colv], x,
                                       mask=mask)
                    return carry2

                lax.fori_loop(0, _D, col_body, 0)

            return carry

        lax.fori_loop(0, _NGRP, fix_body, 0)

    prep(0, 0)

    def step_body(step, carry):
        for b in range(2):
            i = step * 2 + b
            nxt = i + 1

            for d in gather_descs(b):
                d.wait()

            @pl.when(flag_v[pl.ds(0, _L)][0] > 0)
            def _fixup():
                fixup(b)

            write_desc(b, base + i * _C).start()

            @pl.when(nxt < _NCHUNK)
            def _prep():
                @pl.when(i >= 1)
                def _drain():
                    write_desc(1 - b, base).wait()

                prep(nxt, 1 - b)
        return carry

    lax.fori_loop(0, _NCHUNK // 2, step_body, 0)

    write_desc(0, base).wait()
    write_desc(1, base).wait()


_ext_embed = functools.partial(
    pl.kernel,
    out_type=jax.ShapeDtypeStruct((_B_TOTAL, _D), jnp.float32),
    mesh=plsc.VectorSubcoreMesh(core_axis_name="c", subcore_axis_name="s"),
    compiler_params=pltpu.CompilerParams(
        needs_layout_passes=False, use_tc_tiling_on_sc=False),
    scratch_types=[
        pltpu.VMEM((_NUM_NEW, _D), jnp.float32),
        pltpu.VMEM((_C,), jnp.int32),
        pltpu.VMEM((_NDMA, _IDXW), jnp.int32),
        pltpu.VMEM((2, _C, _D), jnp.float32),
        pltpu.VMEM((_L,), jnp.int32),
        pltpu.SemaphoreType.DMA,
        pltpu.SemaphoreType.DMA,
        pltpu.SemaphoreType.DMA,
        pltpu.SemaphoreType.DMA,
    ],
)(_body)


def kernel(input_ids, W_orig, W_new):
    ids = input_ids.reshape(-1).astype(jnp.int32)
    out = _ext_embed(ids, W_orig, W_new)
    return out.reshape(input_ids.shape + (_D,))


# speedup vs baseline: 1.0033x; 1.0033x over previous
import functools

import jax
import jax.numpy as jnp
from jax import lax
from jax.experimental import pallas as pl
from jax.experimental.pallas import tpu as pltpu
from jax.experimental.pallas import tpu_sc as plsc

_NUM_ORIG = 1000000
_NUM_NEW = 1000
_D = 64
_L = 16

_NC = 2
_NS = 16
_NW = _NC * _NS

_B_TOTAL = 16384 * 50
_B_PER_W = _B_TOTAL // _NW
_C = 256
_NCHUNK = _B_PER_W // _C
_IDXW = 32
_NDMA = _C // _IDXW
_NGRP = _C // _L


def _body(ids_hbm, w_orig_hbm, w_new_hbm, out_hbm,
          w_new_v, ids_v, idx_v, rows_v, flag_v, sem_g0, sem_g1, sem_w0,
          sem_w1):
    c_id = lax.axis_index("c")
    s_id = lax.axis_index("s")
    wid = s_id * _NC + c_id
    base = wid * _B_PER_W
    sem_g = (sem_g0, sem_g1)
    sem_w = (sem_w0, sem_w1)

    pltpu.sync_copy(w_new_hbm, w_new_v)

    iota = lax.iota(jnp.int32, _L)

    def gather_descs(b):
        return [
            pltpu.make_async_copy(
                w_orig_hbm.at[idx_v.at[b, j]],
                rows_v.at[b].at[pl.ds(j * _IDXW, _IDXW)],
                sem_g[b],
            )
            for j in range(_NDMA)
        ]

    def write_desc(b, pos):
        return pltpu.make_async_copy(
            rows_v.at[b], out_hbm.at[pl.ds(pos, _C)], sem_w[b])

    def prep(chunk, b):
        pos = base + chunk * _C
        pltpu.sync_copy(ids_hbm.at[pl.ds(pos, _C)], ids_v.at[b])
        flag_v[b, pl.ds(0, _L)] = jnp.zeros((_L,), jnp.int32)

        def safe_body(g, carry):
            idvec = ids_v[b, pl.ds(g * _L, _L)]
            mask = idvec >= _NUM_ORIG
            safe = jnp.where(mask, 0, idvec)
            gpr = _IDXW // _L
            idx_v[b, g // gpr, pl.ds((g % gpr) * _L, _L)] = safe
            plsc.store_scatter(flag_v.at[b], [jnp.full((_L,), 0, jnp.int32)],
                               jnp.full((_L,), 1, jnp.int32), mask=mask)
            return carry

        lax.fori_loop(0, _NGRP, safe_body, 0)
        for d in gather_descs(b):
            d.start()

    def fixup(b):
        def fix_body(g, carry):
            idvec = ids_v[b, pl.ds(g * _L, _L)]
            mask = idvec >= _NUM_ORIG
            has_new = plsc.all_reduce_population_count(mask)[0] > 0

            @pl.when(has_new)
            def _fix():
                new_ids = jnp.where(mask, idvec - _NUM_ORIG, 0)
                row_pos = g * _L + iota

                def col_body(col, carry2):
                    colv = jnp.full((_L,), col, jnp.int32)
                    x = plsc.load_gather(w_new_v, [new_ids, colv], mask=mask)
                    plsc.store_scatter(rows_v.at[b], [row_pos, colv], x,
                                       mask=mask)
                    return carry2

                lax.fori_loop(0, _D, col_body, 0)

            return carry

        lax.fori_loop(0, _NGRP, fix_body, 0)

    prep(0, 0)

    def step_body(step, carry):
        for b in range(2):
            i = step * 2 + b
            nxt = i + 1

            @pl.when(nxt < _NCHUNK)
            def _prep():
                @pl.when(i >= 1)
                def _drain():
                    write_desc(1 - b, base).wait()

                prep(nxt, 1 - b)

            for d in gather_descs(b):
                d.wait()

            @pl.when(flag_v[b, pl.ds(0, _L)][0] > 0)
            def _fixup():
                fixup(b)
            write_desc(b, base + i * _C).start()
        return carry

    lax.fori_loop(0, _NCHUNK // 2, step_body, 0)

    write_desc(0, base).wait()
    write_desc(1, base).wait()


_ext_embed = functools.partial(
    pl.kernel,
    out_type=jax.ShapeDtypeStruct((_B_TOTAL, _D), jnp.float32),
    mesh=plsc.VectorSubcoreMesh(core_axis_name="c", subcore_axis_name="s"),
    compiler_params=pltpu.CompilerParams(
        needs_layout_passes=False, use_tc_tiling_on_sc=False),
    scratch_types=[
        pltpu.VMEM((_NUM_NEW, _D), jnp.float32),
        pltpu.VMEM((2, _C), jnp.int32),
        pltpu.VMEM((2, _NDMA, _IDXW), jnp.int32),
        pltpu.VMEM((2, _C, _D), jnp.float32),
        pltpu.VMEM((2, _L), jnp.int32),
        pltpu.SemaphoreType.DMA,
        pltpu.SemaphoreType.DMA,
        pltpu.SemaphoreType.DMA,
        pltpu.SemaphoreType.DMA,
    ],
)(_body)


def kernel(input_ids, W_orig, W_new):
    ids = input_ids.reshape(-1).astype(jnp.int32)
    out = _ext_embed(ids, W_orig, W_new)
    return out.reshape(input_ids.shape + (_D,))
